# baseline (speedup 1.0000x reference)
import functools

import jax
import jax.numpy as jnp
from jax.experimental import pallas as pl
from jax.experimental.pallas import tpu as pltpu

_BR = 256
_CP = 1024
_NFIX = 512


def _pass_a(x_ref, w_ref, mft_ref, b2_ref, out_ref, gap_ref, wm_ref,
            *, num_classes):
    i = pl.program_id(0)

    @pl.when(i == 0)
    def _precompute():
        wm_ref[...] = jnp.dot(w_ref[...], mft_ref[...])

    scores = jnp.dot(x_ref[...], wm_ref[...])
    d2 = b2_ref[...] - 2.0 * scores
    col = jax.lax.broadcasted_iota(jnp.int32, d2.shape, 1)
    d2 = jnp.where(col < num_classes, d2, jnp.inf)
    rowmin = jnp.min(d2, axis=1, keepdims=True)
    cand = jnp.where(d2 == rowmin, col, d2.shape[1])
    idx = jnp.min(cand, axis=1, keepdims=True)
    runner = jnp.min(jnp.where(col == idx, jnp.inf, d2), axis=1, keepdims=True)
    gap_ref[...] = runner - rowmin
    ocol = jax.lax.broadcasted_iota(jnp.int32, out_ref.shape, 1)
    out_ref[...] = (ocol == idx).astype(jnp.float32)


def _pass_b(xf_ref, w_ref, mft_ref, b2_ref, out_ref, *, num_classes):
    preds = jnp.dot(xf_ref[...], w_ref[...])
    a2 = jnp.sum(preds * preds, axis=1, keepdims=True)
    scores = jnp.dot(preds, mft_ref[...])
    d2 = a2 + b2_ref[...] - 2.0 * scores
    dist = jnp.sqrt(jnp.maximum(d2, 0.0))
    col = jax.lax.broadcasted_iota(jnp.int32, dist.shape, 1)
    dist = jnp.where(col < num_classes, dist, jnp.inf)
    rowmin = jnp.min(dist, axis=1, keepdims=True)
    cand = jnp.where(dist == rowmin, col, dist.shape[1])
    idx = jnp.min(cand, axis=1, keepdims=True)
    ocol = jax.lax.broadcasted_iota(jnp.int32, out_ref.shape, 1)
    out_ref[...] = (ocol == idx).astype(jnp.float32)


def kernel(x, W, mean_features):
    ns, d_in = x.shape
    num_classes, nf = mean_features.shape
    mft = jnp.zeros((nf, _CP), mean_features.dtype).at[:, :num_classes].set(
        mean_features.T)
    b2 = jnp.zeros((1, _CP), jnp.float32).at[:, :num_classes].set(
        jnp.sum(mean_features * mean_features, axis=1)[None, :])

    body_a = functools.partial(_pass_a, num_classes=num_classes)
    out, gaps = pl.pallas_call(
        body_a,
        grid=(ns // _BR,),
        in_specs=[
            pl.BlockSpec((_BR, d_in), lambda i: (i, 0)),
            pl.BlockSpec((d_in, nf), lambda i: (0, 0)),
            pl.BlockSpec((nf, _CP), lambda i: (0, 0)),
            pl.BlockSpec((1, _CP), lambda i: (0, 0)),
        ],
        out_specs=[
            pl.BlockSpec((_BR, num_classes), lambda i: (i, 0)),
            pl.BlockSpec((_BR, 1), lambda i: (i, 0)),
        ],
        out_shape=[
            jax.ShapeDtypeStruct((ns, num_classes), jnp.float32),
            jax.ShapeDtypeStruct((ns, 1), jnp.float32),
        ],
        scratch_shapes=[pltpu.VMEM((d_in, _CP), jnp.float32)],
    )(x, W, mft, b2)

    _, risky = jax.lax.top_k(-gaps[:, 0], _NFIX)
    xf = jnp.take(x, risky, axis=0)

    body_b = functools.partial(_pass_b, num_classes=num_classes)
    fixed = pl.pallas_call(
        body_b,
        grid=(1,),
        in_specs=[
            pl.BlockSpec((_NFIX, d_in), lambda i: (0, 0)),
            pl.BlockSpec((d_in, nf), lambda i: (0, 0)),
            pl.BlockSpec((nf, _CP), lambda i: (0, 0)),
            pl.BlockSpec((1, _CP), lambda i: (0, 0)),
        ],
        out_specs=pl.BlockSpec((_NFIX, num_classes), lambda i: (0, 0)),
        out_shape=jax.ShapeDtypeStruct((_NFIX, num_classes), jnp.float32),
    )(xf, W, mft, b2)

    return out.at[risky].set(fixed)
